# baseline (speedup 1.0000x reference)
import functools

import jax
import jax.numpy as jnp
from jax.experimental import pallas as pl
from jax.experimental.pallas import tpu as pltpu

_B = 8192
_NBINS = 30


def _ece_body(nb, n, x_ref, lab_ref, out_ref, acc_ref):
    i = pl.program_id(0)

    @pl.when(i == 0)
    def _init():
        for g in range(3):
            for b in range(_NBINS):
                acc_ref[g, b] = 0.0

    x = x_ref[...]
    lab = lab_ref[0, 0, :]
    conf = jnp.max(x, axis=1)
    pred = jnp.argmax(x, axis=1).astype(jnp.int32)
    accv = (pred == lab).astype(jnp.float32)
    binv = jnp.clip(jnp.ceil(conf * _NBINS).astype(jnp.int32) - 1, 0, _NBINS - 1)
    rows = jax.lax.iota(jnp.int32, _B) + i * _B
    valid = rows < n
    for b in range(_NBINS):
        m = valid & (binv == b)
        acc_ref[0, b] += jnp.sum(jnp.where(m, 1.0, 0.0))
        acc_ref[1, b] += jnp.sum(jnp.where(m, conf, 0.0))
        acc_ref[2, b] += jnp.sum(jnp.where(m, accv, 0.0))

    @pl.when(i == nb - 1)
    def _fin():
        ece = 0.0
        for b in range(_NBINS):
            cnt = acc_ref[0, b]
            safe = jnp.maximum(cnt, 1.0)
            gap = jnp.abs(acc_ref[1, b] / safe - acc_ref[2, b] / safe)
            gap = jnp.where(cnt > 0.0, gap, 0.0)
            ece += gap * (cnt / n)
        out_ref[...] = jnp.broadcast_to(ece, (1, 1))


def kernel(softmaxes, labels):
    n, c = softmaxes.shape
    nb = pl.cdiv(n, _B)
    npad = nb * _B
    lab_p = jnp.pad(labels, (0, npad - n)).reshape(nb, 1, _B)
    out = pl.pallas_call(
        functools.partial(_ece_body, nb, n),
        grid=(nb,),
        in_specs=[
            pl.BlockSpec((_B, c), lambda i: (i, 0)),
            pl.BlockSpec((1, 1, _B), lambda i: (i, 0, 0)),
        ],
        out_specs=pl.BlockSpec((1, 1), lambda i: (0, 0)),
        out_shape=jax.ShapeDtypeStruct((1, 1), jnp.float32),
        scratch_shapes=[pltpu.SMEM((4, 32), jnp.float32)],
    )(softmaxes, lab_p)
    return out.reshape(1)


# speedup vs baseline: 6.1541x; 6.1541x over previous
import functools

import jax
import jax.numpy as jnp
from jax import lax
from jax.experimental import pallas as pl
from jax.experimental.pallas import tpu as pltpu

_B = 8192
_NBINS = 30


def _ece_body(nb, n, c, x_ref, lab_ref, out_ref, acc_ref):
    i = pl.program_id(0)

    @pl.when(i == 0)
    def _init():
        acc_ref[...] = jnp.zeros((3, 32), jnp.float32)

    x = x_ref[...]
    eye = (lax.broadcasted_iota(jnp.int32, (c, c), 0)
           == lax.broadcasted_iota(jnp.int32, (c, c), 1)).astype(jnp.float32)
    xt = lax.dot_general(eye, x, (((1,), (1,)), ((), ())),
                         preferred_element_type=jnp.float32)

    conf = jnp.max(xt, axis=0, keepdims=True)
    sub_iota = lax.broadcasted_iota(jnp.int32, (c, _B), 0)
    pred = jnp.min(jnp.where(xt == conf, sub_iota, c), axis=0,
                   keepdims=True)
    lab = lab_ref[0]
    accv = (pred == lab).astype(jnp.float32)
    binv = jnp.clip(jnp.ceil(conf * _NBINS).astype(jnp.int32) - 1,
                    0, _NBINS - 1)
    rows = lax.broadcasted_iota(jnp.int32, (1, _B), 1) + i * _B
    valid = rows < n

    bin_iota = lax.broadcasted_iota(jnp.int32, (32, _B), 0)
    m = ((binv == bin_iota) & valid).astype(jnp.float32)
    y = jnp.concatenate(
        [valid.astype(jnp.float32),
         jnp.where(valid, conf, 0.0),
         jnp.where(valid, accv, 0.0)], axis=0)
    s = lax.dot_general(y, m, (((1,), (1,)), ((), ())),
                        preferred_element_type=jnp.float32)
    acc_ref[...] += s

    @pl.when(i == nb - 1)
    def _fin():
        stats = acc_ref[...]
        cnt = stats[0:1, :]
        safe = jnp.maximum(cnt, 1.0)
        gap = jnp.abs(stats[1:2, :] / safe - stats[2:3, :] / safe)
        gap = jnp.where(cnt > 0.0, gap, 0.0)
        ece = jnp.sum(gap * cnt) / n
        out_ref[...] = jnp.broadcast_to(ece, (1, 1))


def kernel(softmaxes, labels):
    n, c = softmaxes.shape
    nb = pl.cdiv(n, _B)
    npad = nb * _B
    lab_p = jnp.pad(labels, (0, npad - n)).reshape(nb, 1, _B)
    out = pl.pallas_call(
        functools.partial(_ece_body, nb, n, c),
        grid=(nb,),
        in_specs=[
            pl.BlockSpec((_B, c), lambda i: (i, 0)),
            pl.BlockSpec((1, 1, _B), lambda i: (i, 0, 0)),
        ],
        out_specs=pl.BlockSpec((1, 1), lambda i: (0, 0)),
        out_shape=jax.ShapeDtypeStruct((1, 1), jnp.float32),
        scratch_shapes=[pltpu.VMEM((3, 32), jnp.float32)],
    )(softmaxes, lab_p)
    return out.reshape(1)


# speedup vs baseline: 6.2159x; 1.0100x over previous
import functools

import jax
import jax.numpy as jnp
from jax import lax
from jax.experimental import pallas as pl
from jax.experimental.pallas import tpu as pltpu

_B = 8192
_NBINS = 30


def _ece_body(nb, n, c, x_ref, lab_ref, out_ref, acc_ref):
    i = pl.program_id(0)

    @pl.when(i == 0)
    def _init():
        acc_ref[...] = jnp.zeros((3, 32), jnp.float32)

    x = x_ref[...]
    xt = x.T

    conf = jnp.max(xt, axis=0, keepdims=True)
    sub_iota = lax.broadcasted_iota(jnp.int32, (c, _B), 0)
    pred = jnp.min(jnp.where(xt == conf, sub_iota, c), axis=0,
                   keepdims=True)
    lab = lab_ref[0]
    accv = (pred == lab).astype(jnp.float32)
    binv = jnp.clip(jnp.ceil(conf * _NBINS).astype(jnp.int32) - 1,
                    0, _NBINS - 1)
    rows = lax.broadcasted_iota(jnp.int32, (1, _B), 1) + i * _B
    valid = rows < n

    bin_iota = lax.broadcasted_iota(jnp.int32, (32, _B), 0)
    m = ((binv == bin_iota) & valid).astype(jnp.float32)
    y = jnp.concatenate(
        [valid.astype(jnp.float32),
         jnp.where(valid, conf, 0.0),
         jnp.where(valid, accv, 0.0)], axis=0)
    s = lax.dot_general(y, m, (((1,), (1,)), ((), ())),
                        preferred_element_type=jnp.float32)
    acc_ref[...] += s

    @pl.when(i == nb - 1)
    def _fin():
        stats = acc_ref[...]
        cnt = stats[0:1, :]
        safe = jnp.maximum(cnt, 1.0)
        gap = jnp.abs(stats[1:2, :] / safe - stats[2:3, :] / safe)
        gap = jnp.where(cnt > 0.0, gap, 0.0)
        ece = jnp.sum(gap * cnt) / n
        out_ref[...] = jnp.broadcast_to(ece, (1, 1))


def kernel(softmaxes, labels):
    n, c = softmaxes.shape
    nb = pl.cdiv(n, _B)
    npad = nb * _B
    lab_p = jnp.pad(labels, (0, npad - n)).reshape(nb, 1, _B)
    out = pl.pallas_call(
        functools.partial(_ece_body, nb, n, c),
        grid=(nb,),
        in_specs=[
            pl.BlockSpec((_B, c), lambda i: (i, 0)),
            pl.BlockSpec((1, 1, _B), lambda i: (i, 0, 0)),
        ],
        out_specs=pl.BlockSpec((1, 1), lambda i: (0, 0)),
        out_shape=jax.ShapeDtypeStruct((1, 1), jnp.float32),
        scratch_shapes=[pltpu.VMEM((3, 32), jnp.float32)],
    )(softmaxes, lab_p)
    return out.reshape(1)
